# Initial kernel scaffold; baseline (speedup 1.0000x reference)
import jax
import jax.numpy as jnp
from jax.experimental import pallas as pl


def kernel(idx, adjacency_matrix, node_embeddings, label, W1, b1, W2, b2, Wc, bc):
    raise NotImplementedError("write your pallas kernel here")


# speedup vs baseline: 5.3583x; 5.3583x over previous
import functools

import jax
import jax.numpy as jnp
from jax import lax
from jax.experimental import pallas as pl
from jax.experimental.pallas import tpu as pltpu
from jax.experimental.pallas import tpu_sc as plsc

N_NODES = 10000
N_PAIRS = 640000
HIDDEN = 768
DIM = 64

NC = 2
NS = 16
LANES = 16
NW = NC * NS
PAIRS_PER_W = N_PAIRS // NW
CHUNK = 10000
STEPS = CHUNK // LANES


def _dense_body(x_ref, w1_ref, b1_ref, w2_ref, b2_ref, wc_ref, b4_ref, out_ref):
    h = jnp.dot(x_ref[...], w1_ref[...], preferred_element_type=jnp.float32)
    h = jnp.maximum(h + b1_ref[...], 0.0)
    h = jnp.dot(h, w2_ref[...], preferred_element_type=jnp.float32)
    h = jnp.maximum(h + b2_ref[...], 0.0)
    out_ref[...] = jnp.dot(h, wc_ref[...], preferred_element_type=jnp.float32) + b4_ref[...]


def _dense_table(x, w1, b1, w2, b2, wc4, b4):
    rows = 1000
    grid = N_NODES // rows
    return pl.pallas_call(
        _dense_body,
        grid=(grid,),
        in_specs=[
            pl.BlockSpec((rows, HIDDEN), lambda i: (i, 0)),
            pl.BlockSpec((HIDDEN, DIM), lambda i: (0, 0)),
            pl.BlockSpec((1, DIM), lambda i: (0, 0)),
            pl.BlockSpec((DIM, DIM), lambda i: (0, 0)),
            pl.BlockSpec((1, DIM), lambda i: (0, 0)),
            pl.BlockSpec((DIM, 4), lambda i: (0, 0)),
            pl.BlockSpec((1, 4), lambda i: (0, 0)),
        ],
        out_specs=pl.BlockSpec((rows, 4), lambda i: (i, 0)),
        out_shape=jax.ShapeDtypeStruct((N_NODES, 4), jnp.float32),
    )(x, w1, b1, w2, b2, wc4, b4)


def _pairs_body(idx_hbm, lab_hbm, tab_hbm, out_hbm, marg_hbm,
                tab_v, idx_v, lab_v, out_v, marg_v):
    wid = lax.axis_index("c") * NS + lax.axis_index("s")
    base = wid * PAIRS_PER_W
    pltpu.sync_copy(tab_hbm, tab_v)
    iota = lax.iota(jnp.int32, LANES)

    for c in range(PAIRS_PER_W // CHUNK):
        off = base + c * CHUNK
        pltpu.sync_copy(idx_hbm.at[pl.ds(off * 2, CHUNK * 2)], idx_v)
        pltpu.sync_copy(lab_hbm.at[pl.ds(off, CHUNK)], lab_v)

        def step(j, _):
            q = j * LANES + iota
            i0 = plsc.load_gather(idx_v, [q * 2])
            i1 = plsc.load_gather(idx_v, [q * 2 + 1])
            a0 = plsc.load_gather(tab_v, [i0 * 4])
            a1 = plsc.load_gather(tab_v, [i0 * 4 + 1])
            r0 = plsc.load_gather(tab_v, [i1 * 4 + 2])
            r1 = plsc.load_gather(tab_v, [i1 * 4 + 3])
            l0 = a0 + r0
            l1 = a1 + r1
            plsc.store_scatter(out_v, [q * 2], l0)
            plsc.store_scatter(out_v, [q * 2 + 1], l1)
            lab = lab_v[pl.ds(j * LANES, LANES)]
            sgn = (2 * lab - 1).astype(jnp.float32)
            marg_v[pl.ds(j * LANES, LANES)] = (l1 - l0) * sgn
            return 0

        lax.fori_loop(0, STEPS, step, 0)
        pltpu.sync_copy(out_v, out_hbm.at[pl.ds(off * 2, CHUNK * 2)])
        pltpu.sync_copy(marg_v, marg_hbm.at[pl.ds(off, CHUNK)])


_pairs_call = pl.kernel(
    _pairs_body,
    out_type=(
        jax.ShapeDtypeStruct((N_PAIRS * 2,), jnp.float32),
        jax.ShapeDtypeStruct((N_PAIRS,), jnp.float32),
    ),
    mesh=plsc.VectorSubcoreMesh(
        core_axis_name="c", subcore_axis_name="s", num_cores=NC, num_subcores=NS
    ),
    scratch_types=(
        pltpu.VMEM((N_NODES * 4,), jnp.float32),
        pltpu.VMEM((CHUNK * 2,), jnp.int32),
        pltpu.VMEM((CHUNK,), jnp.int32),
        pltpu.VMEM((CHUNK * 2,), jnp.float32),
        pltpu.VMEM((CHUNK,), jnp.float32),
    ),
    compiler_params=pltpu.CompilerParams(needs_layout_passes=False),
)


def _loss_body(t_ref, out_ref):
    x = -t_ref[...]
    sp = jnp.maximum(x, 0.0) + jnp.log1p(jnp.exp(-jnp.abs(x)))
    out_ref[0, 0] = jnp.sum(sp) / N_PAIRS


def _loss(marg):
    return pl.pallas_call(
        _loss_body,
        in_specs=[pl.BlockSpec((N_PAIRS // 128, 128), lambda: (0, 0))],
        out_specs=pl.BlockSpec(memory_space=pltpu.SMEM),
        out_shape=jax.ShapeDtypeStruct((1, 1), jnp.float32),
    )(marg.reshape(N_PAIRS // 128, 128))


def kernel(idx, adjacency_matrix, node_embeddings, label, W1, b1, W2, b2, Wc, bc):
    del adjacency_matrix
    wc4 = jnp.concatenate([Wc[:DIM], Wc[DIM:]], axis=1)
    b4 = jnp.concatenate([jnp.zeros_like(bc), bc]).reshape(1, 4)
    tab = _dense_table(node_embeddings, W1, b1.reshape(1, DIM), W2,
                       b2.reshape(1, DIM), wc4, b4)
    logits_flat, marg = _pairs_call(idx.reshape(-1), label, tab.reshape(-1))
    loss = _loss(marg)[0, 0]
    return (loss, logits_flat.reshape(N_PAIRS, 2))
